# Initial kernel scaffold; baseline (speedup 1.0000x reference)
import jax
import jax.numpy as jnp
from jax.experimental import pallas as pl

B, T, E = 4, 8192, 768
RATIO = 16
H, D = 4, 16
TOPK = 8
G = T // RATIO
HD = H * D

TBLK_A = 512
TBLK_B = 1024

_EPS = 1e-6
_SCALE = 1.0 / (H * (D ** 0.5))


def _rms_cols(v, m):
    ss = jax.lax.dot_general(v * v, m, (((1,), (0,)), ((), ())),
                             preferred_element_type=jnp.float32)
    return v * jax.lax.rsqrt(ss * (1.0 / D) + _EPS)


def _phase_a(x_ref, w_ref, ape_ref, hm_ref, keys_ref, q_ref):
    x = x_ref[0]
    w = w_ref[...]
    hm = hm_ref[...]
    proj = jax.lax.dot_general(x, w, (((1,), (1,)), ((), ())),
                               preferred_element_type=jnp.float32)
    q = proj[:, :HD]
    k = proj[:, HD:2 * HD]
    g = proj[:, 2 * HD:]
    ng = TBLK_A // RATIO
    g3 = g.reshape(ng, RATIO, HD) + ape_ref[...][None]
    g3 = g3 - jnp.max(g3, axis=1, keepdims=True)
    e = jnp.exp(g3)
    wsm = e / jnp.sum(e, axis=1, keepdims=True)
    kk = (k.reshape(ng, RATIO, HD) * wsm).sum(axis=1)
    keys_ref[0] = _rms_cols(kk, hm)
    q_ref[0] = _rms_cols(q, hm)


def _phase_b(q_ref, keys_ref, mask_ref):
    tb = pl.program_id(1)
    q = q_ref[0]
    keys = keys_ref[0]
    s = jax.lax.dot_general(q, keys, (((1,), (1,)), ((), ())),
                            preferred_element_type=jnp.float32) * _SCALE
    tglob = tb * TBLK_B + jax.lax.broadcasted_iota(jnp.int32, (TBLK_B, G), 0)
    gidx = jax.lax.broadcasted_iota(jnp.int32, (TBLK_B, G), 1)
    causal = (gidx * RATIO + (RATIO - 1)) <= tglob
    neg = jnp.float32(-jnp.inf)
    s = jnp.where(causal, s, neg)
    r = s
    thresh = None
    for _ in range(TOPK):
        thresh = jnp.max(r, axis=-1, keepdims=True)
        r = jnp.where(r == thresh, neg, r)
    mask = (s >= thresh) & causal
    mask_ref[0] = mask.astype(jnp.int8)


def _build(interpret=False):
    a = pl.pallas_call(
        _phase_a,
        grid=(B, T // TBLK_A),
        in_specs=[
            pl.BlockSpec((1, TBLK_A, E), lambda b, t: (b, t, 0)),
            pl.BlockSpec((3 * HD, E), lambda b, t: (0, 0)),
            pl.BlockSpec((RATIO, HD), lambda b, t: (0, 0)),
            pl.BlockSpec((HD, HD), lambda b, t: (0, 0)),
        ],
        out_specs=[
            pl.BlockSpec((1, TBLK_A // RATIO, HD), lambda b, t: (b, t, 0)),
            pl.BlockSpec((1, TBLK_A, HD), lambda b, t: (b, t, 0)),
        ],
        out_shape=[
            jax.ShapeDtypeStruct((B, G, HD), jnp.float32),
            jax.ShapeDtypeStruct((B, T, HD), jnp.float32),
        ],
        interpret=interpret,
    )
    b = pl.pallas_call(
        _phase_b,
        grid=(B, T // TBLK_B),
        in_specs=[
            pl.BlockSpec((1, TBLK_B, HD), lambda b, t: (b, t, 0)),
            pl.BlockSpec((1, G, HD), lambda b, t: (b, 0, 0)),
        ],
        out_specs=pl.BlockSpec((1, TBLK_B, G), lambda b, t: (b, t, 0)),
        out_shape=jax.ShapeDtypeStruct((B, T, G), jnp.int8),
        interpret=interpret,
    )
    return a, b


_PHASE_A_CALL, _PHASE_B_CALL = _build()


def kernel(x, Wq, Wk, Wg, ape):
    w = jnp.concatenate([Wq, Wk, Wg], axis=0)
    ape2 = ape.reshape(RATIO, HD)
    head_m = jnp.kron(jnp.eye(H, dtype=jnp.float32),
                      jnp.ones((D, D), dtype=jnp.float32))
    keys, q = _PHASE_A_CALL(x, w, ape2, head_m)
    mask_i8 = _PHASE_B_CALL(q, keys)
    group_ends = jnp.minimum(jnp.arange(RATIO - 1, G * RATIO, RATIO), T - 1)
    return (mask_i8.astype(jnp.bool_), group_ends)


# speedup vs baseline: 16.0320x; 16.0320x over previous
import jax
import jax.numpy as jnp
from jax.experimental import pallas as pl

B, T, E = 4, 8192, 768
RATIO = 16
H, D = 4, 16
TOPK = 8
G = T // RATIO
HD = H * D

TBLK_A = 512
TBLK_B = 1024

_EPS = 1e-6
_SCALE = 1.0 / (H * (D ** 0.5))


def _rms_cols(v, m):
    ss = jax.lax.dot_general(v * v, m, (((1,), (0,)), ((), ())),
                             preferred_element_type=jnp.float32,
                             precision=jax.lax.Precision.HIGHEST)
    return v * jax.lax.rsqrt(ss * (1.0 / D) + _EPS)


def _phase_a(x_ref, w_ref, ape_ref, hm_ref, keys_ref, q_ref):
    x = x_ref[0]
    w = w_ref[...]
    hm = hm_ref[...]
    proj = jax.lax.dot_general(x, w, (((1,), (1,)), ((), ())),
                               preferred_element_type=jnp.float32)
    q = proj[:, :HD]
    k = proj[:, HD:2 * HD]
    g = proj[:, 2 * HD:]
    ng = TBLK_A // RATIO
    g3 = g.reshape(ng, RATIO, HD) + ape_ref[...][None]
    g3 = g3 - jnp.max(g3, axis=1, keepdims=True)
    e = jnp.exp(g3)
    wsm = e / jnp.sum(e, axis=1, keepdims=True)
    kk = (k.reshape(ng, RATIO, HD) * wsm).sum(axis=1)
    keys_ref[0] = _rms_cols(kk, hm).astype(jnp.bfloat16)
    q_ref[0] = _rms_cols(q, hm).astype(jnp.bfloat16)


def _phase_b(q_ref, keys_ref, mask_ref):
    tb = pl.program_id(1)
    q = q_ref[0]
    keys = keys_ref[0]
    s = jax.lax.dot_general(q, keys, (((1,), (1,)), ((), ())),
                            preferred_element_type=jnp.float32) * _SCALE
    tglob = tb * TBLK_B + jax.lax.broadcasted_iota(jnp.int32, (TBLK_B, G), 0)
    gidx = jax.lax.broadcasted_iota(jnp.int32, (TBLK_B, G), 1)
    causal = (gidx * RATIO + (RATIO - 1)) <= tglob
    neg = jnp.float32(-jnp.inf)
    s = jnp.where(causal, s, neg)
    r = s
    thresh = None
    for _ in range(TOPK):
        thresh = jnp.max(r, axis=-1, keepdims=True)
        r = jnp.where(r == thresh, neg, r)
    mask = (s >= thresh) & causal
    mask_ref[0] = mask.astype(jnp.int8)


def _build(interpret=False):
    a = pl.pallas_call(
        _phase_a,
        grid=(B, T // TBLK_A),
        in_specs=[
            pl.BlockSpec((1, TBLK_A, E), lambda b, t: (b, t, 0)),
            pl.BlockSpec((3 * HD, E), lambda b, t: (0, 0)),
            pl.BlockSpec((RATIO, HD), lambda b, t: (0, 0)),
            pl.BlockSpec((HD, HD), lambda b, t: (0, 0)),
        ],
        out_specs=[
            pl.BlockSpec((1, TBLK_A // RATIO, HD), lambda b, t: (b, t, 0)),
            pl.BlockSpec((1, TBLK_A, HD), lambda b, t: (b, t, 0)),
        ],
        out_shape=[
            jax.ShapeDtypeStruct((B, G, HD), jnp.bfloat16),
            jax.ShapeDtypeStruct((B, T, HD), jnp.bfloat16),
        ],
        interpret=interpret,
    )
    b = pl.pallas_call(
        _phase_b,
        grid=(B, T // TBLK_B),
        in_specs=[
            pl.BlockSpec((1, TBLK_B, HD), lambda b, t: (b, t, 0)),
            pl.BlockSpec((1, G, HD), lambda b, t: (b, 0, 0)),
        ],
        out_specs=pl.BlockSpec((1, TBLK_B, G), lambda b, t: (b, t, 0)),
        out_shape=jax.ShapeDtypeStruct((B, T, G), jnp.int8),
        interpret=interpret,
    )
    return a, b


_PHASE_A_CALL, _PHASE_B_CALL = _build()


def kernel(x, Wq, Wk, Wg, ape):
    xb = x.astype(jnp.bfloat16)
    w = jnp.concatenate([Wq, Wk, Wg], axis=0).astype(jnp.bfloat16)
    ape2 = ape.reshape(RATIO, HD)
    head_m = jnp.kron(jnp.eye(H, dtype=jnp.float32),
                      jnp.ones((D, D), dtype=jnp.float32))
    keys, q = _PHASE_A_CALL(xb, w, ape2, head_m)
    mask_i8 = _PHASE_B_CALL(q, keys)
    group_ends = jnp.minimum(jnp.arange(RATIO - 1, G * RATIO, RATIO), T - 1)
    return (mask_i8.astype(jnp.bool_), group_ends)


# speedup vs baseline: 18.7953x; 1.1724x over previous
import jax
import jax.numpy as jnp
from jax.experimental import pallas as pl

B, T, E = 4, 8192, 768
RATIO = 16
H, D = 4, 16
TOPK = 8
G = T // RATIO
HD = H * D

TBLK_A = 512
TBLK_B = 1024

_EPS = 1e-6
_SCALE = 1.0 / (H * (D ** 0.5))


def _rms_cols(v, m):
    ss = jax.lax.dot_general(v * v, m, (((1,), (0,)), ((), ())),
                             preferred_element_type=jnp.float32,
                             precision=jax.lax.Precision.HIGHEST)
    return v * jax.lax.rsqrt(ss * (1.0 / D) + _EPS)


def _phase_a(x_ref, w_ref, ape_ref, hm_ref, keys_ref, q_ref):
    x = x_ref[0].astype(jnp.bfloat16)
    w = w_ref[...]
    hm = hm_ref[...]
    proj = jax.lax.dot_general(x, w, (((1,), (1,)), ((), ())),
                               preferred_element_type=jnp.float32)
    q = proj[:, :HD]
    k = proj[:, HD:2 * HD]
    g = proj[:, 2 * HD:]
    ng = TBLK_A // RATIO
    g3 = g.reshape(ng, RATIO, HD) + ape_ref[...][None]
    g3 = g3 - jnp.max(g3, axis=1, keepdims=True)
    e = jnp.exp(g3)
    wsm = e / jnp.sum(e, axis=1, keepdims=True)
    kk = (k.reshape(ng, RATIO, HD) * wsm).sum(axis=1)
    keys_ref[0] = _rms_cols(kk, hm).astype(jnp.bfloat16)
    q_ref[0] = _rms_cols(q, hm).astype(jnp.bfloat16)


def _phase_b(q_ref, keys_ref, mask_ref):
    tb = pl.program_id(1)
    q = q_ref[0]
    keys = keys_ref[0]
    s = jax.lax.dot_general(q, keys, (((1,), (1,)), ((), ())),
                            preferred_element_type=jnp.float32) * _SCALE
    tglob = tb * TBLK_B + jax.lax.broadcasted_iota(jnp.int32, (TBLK_B, G), 0)
    gidx = jax.lax.broadcasted_iota(jnp.int32, (TBLK_B, G), 1)
    causal = (gidx * RATIO + (RATIO - 1)) <= tglob
    neg = jnp.float32(-jnp.inf)
    s = jnp.where(causal, s, neg)
    r = s
    thresh = None
    for _ in range(TOPK):
        thresh = jnp.max(r, axis=-1, keepdims=True)
        r = jnp.where(r == thresh, neg, r)
    mask = (s >= thresh) & causal
    mask_ref[0] = mask


def _build(interpret=False):
    a = pl.pallas_call(
        _phase_a,
        grid=(B, T // TBLK_A),
        in_specs=[
            pl.BlockSpec((1, TBLK_A, E), lambda b, t: (b, t, 0)),
            pl.BlockSpec((3 * HD, E), lambda b, t: (0, 0)),
            pl.BlockSpec((RATIO, HD), lambda b, t: (0, 0)),
            pl.BlockSpec((HD, HD), lambda b, t: (0, 0)),
        ],
        out_specs=[
            pl.BlockSpec((1, TBLK_A // RATIO, HD), lambda b, t: (b, t, 0)),
            pl.BlockSpec((1, TBLK_A, HD), lambda b, t: (b, t, 0)),
        ],
        out_shape=[
            jax.ShapeDtypeStruct((B, G, HD), jnp.bfloat16),
            jax.ShapeDtypeStruct((B, T, HD), jnp.bfloat16),
        ],
        interpret=interpret,
    )
    b = pl.pallas_call(
        _phase_b,
        grid=(B, T // TBLK_B),
        in_specs=[
            pl.BlockSpec((1, TBLK_B, HD), lambda b, t: (b, t, 0)),
            pl.BlockSpec((1, G, HD), lambda b, t: (b, 0, 0)),
        ],
        out_specs=pl.BlockSpec((1, TBLK_B, G), lambda b, t: (b, t, 0)),
        out_shape=jax.ShapeDtypeStruct((B, T, G), jnp.bool_),
        interpret=interpret,
    )
    return a, b


_PHASE_A_CALL, _PHASE_B_CALL = _build()


def kernel(x, Wq, Wk, Wg, ape):
    w = jnp.concatenate([Wq, Wk, Wg], axis=0).astype(jnp.bfloat16)
    ape2 = ape.reshape(RATIO, HD)
    head_m = jnp.kron(jnp.eye(H, dtype=jnp.float32),
                      jnp.ones((D, D), dtype=jnp.float32))
    keys, q = _PHASE_A_CALL(x, w, ape2, head_m)
    mask = _PHASE_B_CALL(q, keys)
    group_ends = jnp.minimum(jnp.arange(RATIO - 1, G * RATIO, RATIO), T - 1)
    return (mask, group_ends)


# speedup vs baseline: 20.6847x; 1.1005x over previous
import jax
import jax.numpy as jnp
from jax.experimental import pallas as pl

B, T, E = 4, 8192, 768
RATIO = 16
H, D = 4, 16
TOPK = 8
G = T // RATIO
HD = H * D

TBLK_A = 1024
TBLK_B = 1024

_EPS = 1e-6
_SCALE = 1.0 / (H * (D ** 0.5))


def _rms_cols(v, m):
    ss = jax.lax.dot_general(v * v, m, (((1,), (0,)), ((), ())),
                             preferred_element_type=jnp.float32,
                             precision=jax.lax.Precision.HIGHEST)
    return v * jax.lax.rsqrt(ss * (1.0 / D) + _EPS)


def _phase_a(x_ref, w_ref, ape_ref, hm_ref, keys_ref, q_ref):
    x = x_ref[0].astype(jnp.bfloat16)
    w = w_ref[...]
    hm = hm_ref[...]
    proj = jax.lax.dot_general(x, w, (((1,), (1,)), ((), ())),
                               preferred_element_type=jnp.float32)
    q = proj[:, :HD]
    k = proj[:, HD:2 * HD]
    g = proj[:, 2 * HD:]
    ng = TBLK_A // RATIO
    g3 = g.reshape(ng, RATIO, HD) + ape_ref[...][None]
    g3 = g3 - jnp.max(g3, axis=1, keepdims=True)
    e = jnp.exp(g3)
    wsm = e / jnp.sum(e, axis=1, keepdims=True)
    kk = (k.reshape(ng, RATIO, HD) * wsm).sum(axis=1)
    keys_ref[0] = _rms_cols(kk, hm).astype(jnp.bfloat16)
    q_ref[0] = _rms_cols(q, hm).astype(jnp.bfloat16)


def _phase_b(q_ref, keys_ref, mask_ref):
    tb = pl.program_id(1)
    q = q_ref[0]
    keys = keys_ref[0]
    s = jax.lax.dot_general(q, keys, (((1,), (1,)), ((), ())),
                            preferred_element_type=jnp.float32) * _SCALE
    tglob = tb * TBLK_B + jax.lax.broadcasted_iota(jnp.int32, (TBLK_B, G), 0)
    gidx = jax.lax.broadcasted_iota(jnp.int32, (TBLK_B, G), 1)
    causal = (gidx * RATIO + (RATIO - 1)) <= tglob
    neg = jnp.float32(-jnp.inf)
    s = jnp.where(causal, s, neg)
    r = s
    thresh = None
    for i in range(TOPK):
        thresh = jnp.max(r, axis=-1, keepdims=True)
        if i < TOPK - 1:
            r = jnp.where(r == thresh, neg, r)
    mask = (s >= thresh) & causal
    mask_ref[0] = mask


def _build(interpret=False):
    a = pl.pallas_call(
        _phase_a,
        grid=(B, T // TBLK_A),
        in_specs=[
            pl.BlockSpec((1, TBLK_A, E), lambda b, t: (b, t, 0)),
            pl.BlockSpec((3 * HD, E), lambda b, t: (0, 0)),
            pl.BlockSpec((RATIO, HD), lambda b, t: (0, 0)),
            pl.BlockSpec((HD, HD), lambda b, t: (0, 0)),
        ],
        out_specs=[
            pl.BlockSpec((1, TBLK_A // RATIO, HD), lambda b, t: (b, t, 0)),
            pl.BlockSpec((1, TBLK_A, HD), lambda b, t: (b, t, 0)),
        ],
        out_shape=[
            jax.ShapeDtypeStruct((B, G, HD), jnp.bfloat16),
            jax.ShapeDtypeStruct((B, T, HD), jnp.bfloat16),
        ],
        interpret=interpret,
    )
    b = pl.pallas_call(
        _phase_b,
        grid=(B, T // TBLK_B),
        in_specs=[
            pl.BlockSpec((1, TBLK_B, HD), lambda b, t: (b, t, 0)),
            pl.BlockSpec((1, G, HD), lambda b, t: (b, 0, 0)),
        ],
        out_specs=pl.BlockSpec((1, TBLK_B, G), lambda b, t: (b, t, 0)),
        out_shape=jax.ShapeDtypeStruct((B, T, G), jnp.bool_),
        interpret=interpret,
    )
    return a, b


_PHASE_A_CALL, _PHASE_B_CALL = _build()


def kernel(x, Wq, Wk, Wg, ape):
    w = jnp.concatenate([Wq, Wk, Wg], axis=0).astype(jnp.bfloat16)
    ape2 = ape.reshape(RATIO, HD)
    head_m = jnp.kron(jnp.eye(H, dtype=jnp.float32),
                      jnp.ones((D, D), dtype=jnp.float32))
    keys, q = _PHASE_A_CALL(x, w, ape2, head_m)
    mask = _PHASE_B_CALL(q, keys)
    group_ends = jnp.minimum(jnp.arange(RATIO - 1, G * RATIO, RATIO), T - 1)
    return (mask, group_ends)
